# baseline (speedup 1.0000x reference)
import jax
import jax.numpy as jnp
from jax.experimental import pallas as pl
from jax.experimental.pallas import tpu as pltpu


def _add_kernel(x_ref, p_ref, o_ref):
    o_ref[...] = x_ref[...] + p_ref[...]


def kernel(x, pos_table):
    B, S, D = x.shape
    SB = 1024
    grid = (S // SB, B)
    return pl.pallas_call(
        _add_kernel,
        grid=grid,
        in_specs=[
            pl.BlockSpec((1, SB, D), lambda s, b: (b, s, 0)),
            pl.BlockSpec((SB, D), lambda s, b: (s, 0)),
        ],
        out_specs=pl.BlockSpec((1, SB, D), lambda s, b: (b, s, 0)),
        out_shape=jax.ShapeDtypeStruct(x.shape, x.dtype),
        compiler_params=pltpu.CompilerParams(
            dimension_semantics=("parallel", "parallel"),
        ),
    )(x, pos_table[:S])
